# baseline (speedup 1.0000x reference)
import functools
import jax
import jax.numpy as jnp
from jax import lax
from jax.experimental import pallas as pl
from jax.experimental.pallas import tpu as pltpu
from jax.experimental.pallas import tpu_sc as plsc

B = 32
C = 3
H = 512
W = 512
L = 16
R = 32
NCH = H // R
VPR = W // L


def _sc_body(img, msk, wpad, bpad, out, ximg, xmsk, wv, bv):
    cid = lax.axis_index("c")
    sid = lax.axis_index("s")
    bb = sid * 2 + cid

    pltpu.sync_copy(wpad, wv)
    pltpu.sync_copy(bpad, bv)

    pooled = []
    for c in range(C):
        def chunk_body(k, acc, c=c):
            pltpu.sync_copy(img.at[bb, c, pl.ds(k * R, R), :], ximg)

            def row_body(r, acc):
                for j in range(VPR):
                    acc = acc + ximg[r, pl.ds(j * L, L)]
                return acc

            return lax.fori_loop(0, R, row_body, acc)

        acc = lax.fori_loop(0, NCH, chunk_body, jnp.zeros((L,), jnp.float32))
        pooled.append(jnp.sum(acc) * (1.0 / (H * W)))

    logits = bv[0, :]
    for c in range(C):
        logits = logits + pooled[c] * wv[c, :]
    sig = 1.0 / (1.0 + jnp.exp(-logits))
    lane = lax.iota(jnp.int32, L)
    t = jnp.sum(jnp.where(lane == 3, sig, 0.0))
    cb = [jnp.sum(jnp.where(lane == c, sig, 0.0)) * (1.0 - t) for c in range(C)]

    def chunk_body2(k, carry):
        pltpu.sync_copy(msk.at[bb, pl.ds(k * R, R), :], xmsk)
        for c in range(C):
            pltpu.sync_copy(img.at[bb, c, pl.ds(k * R, R), :], ximg)

            def row_body2(r, carry, c=c):
                for j in range(VPR):
                    sl = pl.ds(j * L, L)
                    x = ximg[r, sl]
                    m = xmsk[r, sl]
                    ximg[r, sl] = jnp.where(m != 0, cb[c] + t * x, x)
                return carry

            lax.fori_loop(0, R, row_body2, 0)
            pltpu.sync_copy(ximg, out.at[bb, c, pl.ds(k * R, R), :])
        return carry

    lax.fori_loop(0, NCH, chunk_body2, 0)


def kernel(image, mask, W_arg, b_arg):
    wpad = jnp.zeros((C, L), jnp.float32).at[:, :4].set(W_arg)
    bpad = jnp.zeros((1, L), jnp.float32).at[0, :4].set(b_arg)
    mesh = plsc.VectorSubcoreMesh(
        core_axis_name="c", subcore_axis_name="s", num_cores=2, num_subcores=16
    )
    f = functools.partial(
        pl.kernel,
        out_type=jax.ShapeDtypeStruct(image.shape, image.dtype),
        mesh=mesh,
        compiler_params=pltpu.CompilerParams(needs_layout_passes=False),
        scratch_types=[
            pltpu.VMEM((R, W), jnp.float32),
            pltpu.VMEM((R, W), jnp.int32),
            pltpu.VMEM((C, L), jnp.float32),
            pltpu.VMEM((1, L), jnp.float32),
        ],
    )(_sc_body)
    return f(image, mask, wpad, bpad)


# speedup vs baseline: 4.6200x; 4.6200x over previous
import jax
import jax.numpy as jnp
from jax.experimental import pallas as pl
from jax.experimental.pallas import tpu as pltpu


def _body(img_ref, msk_ref, w_ref, b_ref, out_ref):
    x = img_ref[0]
    C = 3
    H = x.shape[0] // C
    pooled = jnp.mean(x.reshape(C, H, x.shape[1]), axis=(1, 2))
    logits = jnp.sum(pooled[:, None] * w_ref[...], axis=0) + b_ref[...]
    sig = jax.nn.sigmoid(logits)
    color = sig[:3]
    t = sig[3]
    cb = (color * (1.0 - t))[:, None, None]
    m = (msk_ref[0] != 0)[None, :, :]
    xc = x.reshape(C, H, x.shape[1])
    out_ref[0] = jnp.where(m, cb + t * xc, xc).reshape(x.shape)


def kernel(image, mask, W, b):
    B, C, H, Wd = image.shape
    img2 = image.reshape(B, C * H, Wd)
    out = pl.pallas_call(
        _body,
        grid=(B,),
        in_specs=[
            pl.BlockSpec((1, C * H, Wd), lambda i: (i, 0, 0)),
            pl.BlockSpec((1, H, Wd), lambda i: (i, 0, 0)),
            pl.BlockSpec((C, 4), lambda i: (0, 0)),
            pl.BlockSpec((4,), lambda i: (0,)),
        ],
        out_specs=pl.BlockSpec((1, C * H, Wd), lambda i: (i, 0, 0)),
        out_shape=jax.ShapeDtypeStruct(img2.shape, image.dtype),
        compiler_params=pltpu.CompilerParams(
            dimension_semantics=("arbitrary",),
            vmem_limit_bytes=100 * 1024 * 1024,
        ),
    )(img2, mask, W, b)
    return out.reshape(image.shape)


# speedup vs baseline: 5.0002x; 1.0823x over previous
import jax
import jax.numpy as jnp
from jax.experimental import pallas as pl
from jax.experimental.pallas import tpu as pltpu

_NB = 2


def _body(img_ref, msk_ref, w_ref, b_ref, out_ref):
    C = 3
    x = img_ref[...]
    H = x.shape[1] // C
    x4 = x.reshape(_NB, C, H, x.shape[2])
    pooled = jnp.mean(x4, axis=(2, 3))
    logits = jnp.sum(pooled[:, :, None] * w_ref[...][None], axis=1) + b_ref[...][None]
    sig = jax.nn.sigmoid(logits)
    color = sig[:, :3]
    t = sig[:, 3][:, None, None, None]
    cb = color[:, :, None, None] * (1.0 - t)
    m = (msk_ref[...] != 0)[:, None, :, :]
    out_ref[...] = jnp.where(m, cb + t * x4, x4).reshape(x.shape)


def kernel(image, mask, W, b):
    B, C, H, Wd = image.shape
    img2 = image.reshape(B, C * H, Wd)
    out = pl.pallas_call(
        _body,
        grid=(B // _NB,),
        in_specs=[
            pl.BlockSpec((_NB, C * H, Wd), lambda i: (i, 0, 0)),
            pl.BlockSpec((_NB, H, Wd), lambda i: (i, 0, 0)),
            pl.BlockSpec((C, 4), lambda i: (0, 0)),
            pl.BlockSpec((4,), lambda i: (0,)),
        ],
        out_specs=pl.BlockSpec((_NB, C * H, Wd), lambda i: (i, 0, 0)),
        out_shape=jax.ShapeDtypeStruct(img2.shape, image.dtype),
        compiler_params=pltpu.CompilerParams(
            dimension_semantics=("arbitrary",),
            vmem_limit_bytes=100 * 1024 * 1024,
        ),
    )(img2, mask, W, b)
    return out.reshape(image.shape)


# speedup vs baseline: 5.0339x; 1.0067x over previous
import jax
import jax.numpy as jnp
from jax.experimental import pallas as pl
from jax.experimental.pallas import tpu as pltpu

_NB = 2


def _body(img_ref, msk_ref, w_ref, b_ref, out_ref):
    C = 3
    H = img_ref.shape[1] // C
    pooled = jnp.mean(img_ref[...].reshape(_NB, C, H, -1), axis=(2, 3))
    logits = jnp.sum(pooled[:, :, None] * w_ref[...][None], axis=1) + b_ref[...][None]
    sig = jax.nn.sigmoid(logits)
    color = sig[:, :3]
    t = sig[:, 3][:, None, None, None]
    cb = color[:, :, None, None] * (1.0 - t)
    m = (msk_ref[...] != 0)[:, None, :, :]
    x4 = img_ref[...].reshape(_NB, C, H, -1)
    out_ref[...] = jnp.where(m, cb + t * x4, x4).reshape(img_ref.shape)


def kernel(image, mask, W, b):
    B, C, H, Wd = image.shape
    img2 = image.reshape(B, C * H, Wd)
    out = pl.pallas_call(
        _body,
        grid=(B // _NB,),
        in_specs=[
            pl.BlockSpec((_NB, C * H, Wd), lambda i: (i, 0, 0)),
            pl.BlockSpec((_NB, H, Wd), lambda i: (i, 0, 0)),
            pl.BlockSpec((C, 4), lambda i: (0, 0)),
            pl.BlockSpec((4,), lambda i: (0,)),
        ],
        out_specs=pl.BlockSpec((_NB, C * H, Wd), lambda i: (i, 0, 0)),
        out_shape=jax.ShapeDtypeStruct(img2.shape, image.dtype),
        compiler_params=pltpu.CompilerParams(
            dimension_semantics=("arbitrary",),
            vmem_limit_bytes=100 * 1024 * 1024,
        ),
    )(img2, mask, W, b)
    return out.reshape(image.shape)
